# baseline (speedup 1.0000x reference)
import functools

import jax
import jax.numpy as jnp
import numpy as np
from jax import lax
from jax.experimental import pallas as pl
from jax.experimental.pallas import tpu as pltpu
from jax.experimental.pallas import tpu_sc as plsc

_W = 64
_C = 14
_N = 1024
_E = 11
_ROWS = _N * _C
_K = _W * _W
_BM = 256

_COORD_PAD = np.zeros((2, 16), dtype=np.int32)
_COORD_PAD[:, :_E] = np.array(
    [[12, 12, 8, 7, 12, 9, 10, 2, 1, 3, 4],
     [13, 8, 7, 6, 9, 10, 11, 1, 0, 4, 5]], dtype=np.int32)

_NC, _NS, _L = 2, 16, 16


def _reduce_body(x_ref, val_ref, xc_ref, yc_ref):
    blk = x_ref[...]
    m = jnp.max(blk, axis=(1, 2))
    r = lax.broadcasted_iota(jnp.int32, blk.shape, 1)
    l = lax.broadcasted_iota(jnp.int32, blk.shape, 2)
    flat_pos = (r * 128 + l).astype(jnp.float32)
    hit = jnp.where(blk == m[:, None, None], flat_pos, float(_K))
    idx = jnp.min(hit, axis=(1, 2))
    val_ref[...] = m
    xc_ref[...] = (idx.astype(jnp.int32) % _W).astype(jnp.float32) * (1.0 / _W)
    yc_ref[...] = jnp.round(idx * (1.0 / _W)) * (1.0 / _W)


def _maxpool_keypoints(x):
    x3 = x.reshape(_ROWS, _K // 128, 128)
    out = jax.ShapeDtypeStruct((_ROWS,), jnp.float32)
    return pl.pallas_call(
        _reduce_body,
        grid=(_ROWS // _BM,),
        in_specs=[pl.BlockSpec((_BM, _K // 128, 128), lambda i: (i, 0, 0))],
        out_specs=[pl.BlockSpec((_BM,), lambda i: (i,))] * 3,
        out_shape=[out, out, out],
    )(x3)


def _edge_index_sc(coord):
    mesh = plsc.VectorSubcoreMesh(core_axis_name="c", subcore_axis_name="s")
    rows_per_subcore = _N // _NS

    @functools.partial(
        pl.kernel,
        mesh=mesh,
        out_type=jax.ShapeDtypeStruct((2, _N * _L), jnp.int32),
        scratch_types=[
            pltpu.VMEM((_L,), jnp.int32),
            pltpu.VMEM((rows_per_subcore * _L,), jnp.int32),
            pltpu.SemaphoreType.DMA,
            pltpu.SemaphoreType.DMA,
        ],
    )
    def k(coord_hbm, out_hbm, crow, buf, sem_in, sem_out):
        c = lax.axis_index("c")
        s = lax.axis_index("s")
        pltpu.async_copy(coord_hbm.at[c], crow, sem_in).wait()
        base = s * rows_per_subcore

        @pl.loop(0, rows_per_subcore)
        def _(j):
            buf[pl.ds(j * _L, _L)] = crow[...] + (base + j) * _C

        pltpu.async_copy(
            buf, out_hbm.at[c].at[pl.ds(base * _L, rows_per_subcore * _L)],
            sem_out).wait()

    return k(coord)


def kernel(x):
    value, xc, yc = _maxpool_keypoints(x)
    feature = jnp.stack([value, xc, yc], axis=-1)
    coord = jnp.asarray(_COORD_PAD[:, :_E])
    offsets = jnp.arange(_N, dtype=jnp.int32) * _C
    edge_index = (coord[:, None, :] + offsets[None, :, None]).reshape(2, _N * _E)
    return feature, edge_index


# speedup vs baseline: 8.3346x; 8.3346x over previous
import functools

import jax
import jax.numpy as jnp
import numpy as np
from jax import lax
from jax.experimental import pallas as pl
from jax.experimental.pallas import tpu as pltpu
from jax.experimental.pallas import tpu_sc as plsc

_W = 64
_C = 14
_N = 1024
_E = 11
_ROWS = _N * _C
_K = _W * _W
_BM = 256

_COORD_PAD = np.zeros((2, 16), dtype=np.int32)
_COORD_PAD[:, :_E] = np.array(
    [[12, 12, 8, 7, 12, 9, 10, 2, 1, 3, 4],
     [13, 8, 7, 6, 9, 10, 11, 1, 0, 4, 5]], dtype=np.int32)

_NC, _NS, _L = 2, 16, 16


def _reduce_body(x_ref, val_ref, xc_ref, yc_ref):
    blk = x_ref[...]
    m = jnp.max(blk, axis=(1, 2))
    w_i = lax.broadcasted_iota(jnp.int32, blk.shape, 1)
    h_i = lax.broadcasted_iota(jnp.int32, blk.shape, 2)
    flat_pos = (w_i * _W + h_i).astype(jnp.float32)
    hit = jnp.where(blk == m[:, None, None, :], flat_pos, float(_K))
    idx = jnp.min(hit, axis=(1, 2))
    val_ref[...] = m[:, None, :]
    xc_ref[...] = ((idx.astype(jnp.int32) % _W).astype(jnp.float32)
                   * (1.0 / _W))[:, None, :]
    yc_ref[...] = (jnp.round(idx * (1.0 / _W)) * (1.0 / _W))[:, None, :]


def _maxpool_keypoints(x):
    xt = jnp.transpose(x, (1, 2, 3, 0))
    out = jax.ShapeDtypeStruct((_C, 1, _N), jnp.float32)
    v, xc, yc = pl.pallas_call(
        _reduce_body,
        grid=(_C,),
        in_specs=[pl.BlockSpec((1, _W, _W, _N), lambda i: (i, 0, 0, 0))],
        out_specs=[pl.BlockSpec((1, 1, _N), lambda i: (i, 0, 0))] * 3,
        out_shape=[out, out, out],
    )(xt)
    return v[:, 0], xc[:, 0], yc[:, 0]


def _edge_index_sc(coord):
    mesh = plsc.VectorSubcoreMesh(core_axis_name="c", subcore_axis_name="s")
    rows_per_subcore = _N // _NS

    @functools.partial(
        pl.kernel,
        mesh=mesh,
        out_type=jax.ShapeDtypeStruct((2, _N * _L), jnp.int32),
        scratch_types=[
            pltpu.VMEM((_L,), jnp.int32),
            pltpu.VMEM((rows_per_subcore * _L,), jnp.int32),
            pltpu.SemaphoreType.DMA,
            pltpu.SemaphoreType.DMA,
        ],
    )
    def k(coord_hbm, out_hbm, crow, buf, sem_in, sem_out):
        c = lax.axis_index("c")
        s = lax.axis_index("s")
        pltpu.async_copy(coord_hbm.at[c], crow, sem_in).wait()
        base = s * rows_per_subcore

        @pl.loop(0, rows_per_subcore)
        def _(j):
            buf[pl.ds(j * _L, _L)] = crow[...] + (base + j) * _C

        pltpu.async_copy(
            buf, out_hbm.at[c].at[pl.ds(base * _L, rows_per_subcore * _L)],
            sem_out).wait()

    return k(coord)


def kernel(x):
    value, xc, yc = _maxpool_keypoints(x)
    feature = jnp.stack([value, xc, yc], axis=-1)
    feature = jnp.transpose(feature, (1, 0, 2)).reshape(_ROWS, 3)
    coord = jnp.asarray(_COORD_PAD[:, :_E])
    offsets = jnp.arange(_N, dtype=jnp.int32) * _C
    edge_index = (coord[:, None, :] + offsets[None, :, None]).reshape(2, _N * _E)
    return feature, edge_index


# speedup vs baseline: 8.4097x; 1.0090x over previous
import functools

import jax
import jax.numpy as jnp
import numpy as np
from jax import lax
from jax.experimental import pallas as pl
from jax.experimental.pallas import tpu as pltpu
from jax.experimental.pallas import tpu_sc as plsc

_W = 64
_C = 14
_N = 1024
_E = 11
_ROWS = _N * _C
_K = _W * _W
_BM = 256

_COORD_PAD = np.zeros((2, 16), dtype=np.int32)
_COORD_PAD[:, :_E] = np.array(
    [[12, 12, 8, 7, 12, 9, 10, 2, 1, 3, 4],
     [13, 8, 7, 6, 9, 10, 11, 1, 0, 4, 5]], dtype=np.int32)

_NC, _NS, _L = 2, 16, 16


_NSTREAM = 4
_NB = _N // _NSTREAM


def _reduce_body(*refs):
    x_refs, (val_ref, xc_ref, yc_ref) = refs[:_NSTREAM], refs[_NSTREAM:]
    for s, x_ref in enumerate(x_refs):
        blk = x_ref[...]
        m = jnp.max(blk, axis=(1, 2))
        w_i = lax.broadcasted_iota(jnp.int32, blk.shape, 1)
        h_i = lax.broadcasted_iota(jnp.int32, blk.shape, 2)
        flat_pos = (w_i * _W + h_i).astype(jnp.float32)
        hit = jnp.where(blk == m[:, None, None, :], flat_pos, float(_K))
        idx = jnp.min(hit, axis=(1, 2))
        sl = pl.ds(s * _NB, _NB)
        val_ref[:, :, sl] = m[:, None, :]
        xc_ref[:, :, sl] = ((idx.astype(jnp.int32) % _W).astype(jnp.float32)
                            * (1.0 / _W))[:, None, :]
        yc_ref[:, :, sl] = (jnp.round(idx * (1.0 / _W)) * (1.0 / _W))[:, None, :]


def _maxpool_keypoints(x):
    xt = jnp.transpose(x, (1, 2, 3, 0))
    out = jax.ShapeDtypeStruct((_C, 1, _N), jnp.float32)
    v, xc, yc = pl.pallas_call(
        _reduce_body,
        grid=(_C,),
        in_specs=[
            pl.BlockSpec((1, _W, _W, _NB),
                         functools.partial(lambda s, i: (i, 0, 0, s), s))
            for s in range(_NSTREAM)
        ],
        out_specs=[pl.BlockSpec((1, 1, _N), lambda i: (i, 0, 0))] * 3,
        out_shape=[out, out, out],
    )(*([xt] * _NSTREAM))
    return v[:, 0], xc[:, 0], yc[:, 0]


def _edge_index_sc(coord):
    mesh = plsc.VectorSubcoreMesh(core_axis_name="c", subcore_axis_name="s")
    rows_per_subcore = _N // _NS

    @functools.partial(
        pl.kernel,
        mesh=mesh,
        out_type=jax.ShapeDtypeStruct((2, _N * _L), jnp.int32),
        scratch_types=[
            pltpu.VMEM((_L,), jnp.int32),
            pltpu.VMEM((rows_per_subcore * _L,), jnp.int32),
            pltpu.SemaphoreType.DMA,
            pltpu.SemaphoreType.DMA,
        ],
    )
    def k(coord_hbm, out_hbm, crow, buf, sem_in, sem_out):
        c = lax.axis_index("c")
        s = lax.axis_index("s")
        pltpu.async_copy(coord_hbm.at[c], crow, sem_in).wait()
        base = s * rows_per_subcore

        @pl.loop(0, rows_per_subcore)
        def _(j):
            buf[pl.ds(j * _L, _L)] = crow[...] + (base + j) * _C

        pltpu.async_copy(
            buf, out_hbm.at[c].at[pl.ds(base * _L, rows_per_subcore * _L)],
            sem_out).wait()

    return k(coord)


def kernel(x):
    value, xc, yc = _maxpool_keypoints(x)
    feature = jnp.stack([value, xc, yc], axis=-1)
    feature = jnp.transpose(feature, (1, 0, 2)).reshape(_ROWS, 3)
    coord = jnp.asarray(_COORD_PAD[:, :_E])
    offsets = jnp.arange(_N, dtype=jnp.int32) * _C
    edge_index = (coord[:, None, :] + offsets[None, :, None]).reshape(2, _N * _E)
    return feature, edge_index
